# baseline (speedup 1.0000x reference)
import functools

import jax
import jax.numpy as jnp
from jax import lax
from jax.experimental import pallas as pl
from jax.experimental.pallas import tpu as pltpu
from jax.experimental.pallas import tpu_sc as plsc

X_POS = 5
CTX = 77
D = 512
B = 4096

NW = 32
ROWS_PER_W = B // NW
GROUP = 16


@functools.partial(
    pl.kernel,
    mesh=plsc.VectorSubcoreMesh(core_axis_name="c", subcore_axis_name="s"),
    out_type=jax.ShapeDtypeStruct((B, CTX, D), jnp.float32),
    compiler_params=pltpu.CompilerParams(use_tc_tiling_on_sc=True),
    scratch_types=[
        pltpu.VMEM((CTX,), jnp.int32),
        pltpu.VMEM((GROUP, D), jnp.float32),
        pltpu.VMEM((1, CTX, D), jnp.float32),
        pltpu.VMEM((1, CTX, D), jnp.float32),
        pltpu.SemaphoreType.DMA,
        pltpu.SemaphoreType.DMA,
        pltpu.SemaphoreType.DMA,
    ],
)
def _sc_compose(ids_hbm, s_hbm, table_hbm, out_hbm,
                idx_v, schunk, tmpl_a, tmpl_b, gsem, ssem, osem):
    wid = lax.axis_index("s") * 2 + lax.axis_index("c")
    base = wid * ROWS_PER_W
    tmpls = (tmpl_a, tmpl_b)

    pltpu.sync_copy(ids_hbm, idx_v)
    ga = pltpu.make_async_copy(table_hbm.at[idx_v], tmpl_a.at[0], gsem)
    gb = pltpu.make_async_copy(table_hbm.at[idx_v], tmpl_b.at[0], gsem)
    ga.start()
    gb.start()
    ga.wait()
    gb.wait()

    def plane_copy(slot, row):
        return pltpu.make_async_copy(
            tmpls[slot], out_hbm.at[pl.ds(base + row, 1)], osem)

    def group(g, c):
        sc = pltpu.make_async_copy(
            s_hbm.at[pl.ds(base + g * GROUP, GROUP)], schunk, ssem)
        sc.start()
        sc.wait()
        for r8 in range(GROUP):
            row = g * GROUP + r8
            slot = r8 % 2

            @pl.when(row >= 2)
            def _():
                plane_copy(slot, row - 2).wait()

            for cch in range(D // 16):
                tmpls[slot][0, X_POS, pl.ds(cch * 16, 16)] = (
                    schunk[r8, pl.ds(cch * 16, 16)])
            plane_copy(slot, row).start()
        return c

    lax.fori_loop(0, ROWS_PER_W // GROUP, group, 0)

    plane_copy(0, ROWS_PER_W - 2).wait()
    plane_copy(1, ROWS_PER_W - 1).wait()


def _tok_body(tok_ref, tok_out_ref):
    b = tok_out_ref.shape[0]
    tok_out_ref[...] = jnp.broadcast_to(tok_ref[...], (b, CTX))


@jax.jit
def kernel(s_star, tokenized_composed, table):
    b = s_star.shape[0]
    ids = tokenized_composed.reshape(CTX)

    prompts = _sc_compose(ids, s_star.astype(jnp.float32), table)

    tokenized = pl.pallas_call(
        _tok_body,
        in_specs=[pl.BlockSpec((1, CTX), lambda: (0, 0))],
        out_specs=pl.BlockSpec((b, CTX), lambda: (0, 0)),
        out_shape=jax.ShapeDtypeStruct((b, CTX), jnp.int32),
    )(tokenized_composed)

    return (prompts, tokenized)


# speedup vs baseline: 1.0624x; 1.0624x over previous
import jax
import jax.numpy as jnp
from jax.experimental import pallas as pl
from jax.experimental.pallas import tpu as pltpu

X_POS = 5
CTX = 77
D = 512
BB = 64
K = 4


def _body(ids_ref, table_ref, s_ref, tok_ref, out_ref, tok_out_ref,
          emb_scr, svmem, gsem, ssem, o0, o1, o2, o3, *bufs):
    osems = (o0, o1, o2, o3)
    b = tok_out_ref.shape[0]
    nb = b // BB
    nj = nb // K

    s_stage = pltpu.make_async_copy(s_ref, svmem, ssem)
    s_stage.start()

    def gstart(k, c):
        pltpu.make_async_copy(
            table_ref.at[pl.ds(ids_ref[k], 1)],
            emb_scr.at[pl.ds(k, 1)],
            gsem,
        ).start()
        return c

    jax.lax.fori_loop(0, CTX, gstart, 0)

    def gwait(k, c):
        pltpu.make_async_copy(
            table_ref.at[pl.ds(ids_ref[k], 1)],
            emb_scr.at[pl.ds(k, 1)],
            gsem,
        ).wait()
        return c

    jax.lax.fori_loop(0, CTX, gwait, 0)

    def tile_copy(k, j):
        return pltpu.make_async_copy(
            bufs[k], out_ref.at[pl.ds((j * K + k) * BB, BB)], osems[k])

    emb = emb_scr[...]
    s_stage.wait()
    for k in range(K):
        bufs[k][...] = jnp.broadcast_to(emb[None], (BB, CTX, D))
        bufs[k][:, X_POS, :] = svmem[pl.ds(k * BB, BB), :]
        tile_copy(k, 0).start()

    def step(j, c):
        for k in range(K):
            tile_copy(k, j - 1).wait()
            i = j * K + k
            bufs[k][:, X_POS, :] = svmem[pl.ds(i * BB, BB), :]
            tile_copy(k, j).start()
        return c

    jax.lax.fori_loop(1, nj, step, 0)

    tok_out_ref[...] = jnp.broadcast_to(tok_ref[...], (b, CTX))

    for k in range(K):
        tile_copy(k, nj - 1).wait()


@jax.jit
def kernel(s_star, tokenized_composed, table):
    b = s_star.shape[0]
    ids = tokenized_composed.reshape(CTX)

    prompts, tokenized = pl.pallas_call(
        _body,
        in_specs=[
            pl.BlockSpec(memory_space=pltpu.MemorySpace.SMEM),
            pl.BlockSpec(memory_space=pltpu.MemorySpace.HBM),
            pl.BlockSpec(memory_space=pltpu.MemorySpace.HBM),
            pl.BlockSpec((1, CTX), lambda: (0, 0)),
        ],
        out_specs=[
            pl.BlockSpec(memory_space=pltpu.MemorySpace.HBM),
            pl.BlockSpec((b, CTX), lambda: (0, 0)),
        ],
        out_shape=[
            jax.ShapeDtypeStruct((b, CTX, D), jnp.float32),
            jax.ShapeDtypeStruct((b, CTX), jnp.int32),
        ],
        scratch_shapes=[
            pltpu.VMEM((CTX, D), jnp.float32),
            pltpu.VMEM((b, D), jnp.float32),
            pltpu.SemaphoreType.DMA,
            pltpu.SemaphoreType.DMA,
        ] + [pltpu.SemaphoreType.DMA for _ in range(K)]
          + [pltpu.VMEM((BB, CTX, D), jnp.float32) for _ in range(K)],
    )(ids, table, s_star.astype(jnp.float32), tokenized_composed)

    return (prompts, tokenized)
